# Initial kernel scaffold; baseline (speedup 1.0000x reference)
import jax
import jax.numpy as jnp
from jax.experimental import pallas as pl


def kernel(x, edge_index, batch, c1_W1, c1_b1, c1_W2, c1_b2, c2_W1, c2_b1, c2_W2, c2_b2, c3_W1, c3_b1, c3_W2, c3_b2, proj_W1, proj_b1, proj_W2, proj_b2, head_W1, head_b1, head_W2, head_b2):
    raise NotImplementedError("write your pallas kernel here")


# speedup vs baseline: 1.9990x; 1.9990x over previous
import functools

import jax
import jax.numpy as jnp
from jax import lax
from jax.experimental import pallas as pl
from jax.experimental.pallas import tpu as pltpu
from jax.experimental.pallas import tpu_sc as plsc

N = 100000
E = 1600000
H = 32
NT = 32
NPT = N // NT
FC = 4000
QUANT = 1024
RING = 8192
GC = 128
EP = E + NT * QUANT
BM = 1536
NEG = -3.0e38
THRESH = -1.0e38

_mesh = functools.partial(
    plsc.VectorSubcoreMesh, core_axis_name="c", subcore_axis_name="s")


def _wid():
    return lax.axis_index("s") * 2 + lax.axis_index("c")


def _count_edges(dst):
    def body(dst_hbm, cnt_hbm, dbuf, tmp):
        wid = _wid()
        base = wid * NPT
        lo = jnp.full((16,), base, jnp.int32)
        hi = jnp.full((16,), base + NPT, jnp.int32)

        def chunk(ci, acc):
            pltpu.sync_copy(dst_hbm.at[pl.ds(pl.multiple_of(ci * FC, FC), FC)], dbuf)

            def vec(vi, a):
                d = dbuf[pl.ds(vi * 16, 16)]
                m = (d >= lo) & (d < hi)
                return a + jnp.where(m, 1, 0)

            return lax.fori_loop(0, FC // 16, vec, acc)

        acc = lax.fori_loop(0, E // FC, chunk, jnp.zeros((16,), jnp.int32))
        tmp[...] = acc
        pltpu.sync_copy(tmp, cnt_hbm.at[wid])

    f = pl.kernel(
        body,
        out_type=jax.ShapeDtypeStruct((NT, 16), jnp.int32),
        mesh=_mesh(),
        compiler_params=pltpu.CompilerParams(needs_layout_passes=False, use_tc_tiling_on_sc=False),
        scratch_types=[
            pltpu.VMEM((FC,), jnp.int32),
            pltpu.VMEM((16,), jnp.int32),
        ],
    )
    return f(dst)


def _bucket_edges(src, dst, starts48):
    def body(src_hbm, dst_hbm, starts_hbm, bsrc_hbm, bdst_hbm,
             sbuf, dbuf, rings, ringd, stv):
        wid = _wid()
        base = wid * NPT
        lo = jnp.full((16,), base, jnp.int32)
        hi = jnp.full((16,), base + NPT, jnp.int32)
        sent = jnp.full((16,), base + NPT, jnp.int32)
        zero16 = jnp.zeros((16,), jnp.int32)
        lane = lax.iota(jnp.int32, 16)
        pltpu.sync_copy(starts_hbm, stv)
        st = stv[pl.ds(wid, 16)][0]

        def flush_while(cur, flushed):
            def cond(f):
                return cur - f >= QUANT

            def fbody(f):
                off = pl.multiple_of(f & (RING - 1), QUANT)
                dsto = pl.multiple_of(st + f, QUANT)
                pltpu.sync_copy(ringd.at[pl.ds(off, QUANT)],
                                bdst_hbm.at[pl.ds(dsto, QUANT)])
                pltpu.sync_copy(rings.at[pl.ds(off, QUANT)],
                                bsrc_hbm.at[pl.ds(dsto, QUANT)])
                return f + QUANT

            return lax.while_loop(cond, fbody, flushed)

        def chunk(ci, carry):
            curv, flushed = carry
            co = pl.multiple_of(ci * FC, FC)
            pltpu.sync_copy(src_hbm.at[pl.ds(co, FC)], sbuf)
            pltpu.sync_copy(dst_hbm.at[pl.ds(co, FC)], dbuf)

            def vec(vi, cv):
                d = dbuf[pl.ds(vi * 16, 16)]
                s = sbuf[pl.ds(vi * 16, 16)]
                m = (d >= lo) & (d < hi)
                csum = plsc.cumsum(jnp.where(m, 1, 0))
                pos = (cv + csum - 1) & (RING - 1)
                plsc.store_scatter(ringd, [pos], d, mask=m)
                plsc.store_scatter(rings, [pos], s, mask=m)
                return cv + csum[15]

            curv = lax.fori_loop(0, FC // 16, vec, curv)
            flushed = flush_while(curv[0], flushed)
            return curv, flushed

        curv, flushed = lax.fori_loop(
            0, E // FC, chunk, (jnp.zeros((16,), jnp.int32), jnp.int32(0)))

        tgt = ((curv + (QUANT - 1)) >> 10) << 10
        for j in range(QUANT // 16):
            pos = curv + j * 16 + lane
            m = pos < tgt
            plsc.store_scatter(ringd, [pos & (RING - 1)], sent, mask=m)
            plsc.store_scatter(rings, [pos & (RING - 1)], zero16, mask=m)
        flushed = flush_while(tgt[0], flushed)

    f = pl.kernel(
        body,
        out_type=(jax.ShapeDtypeStruct((EP,), jnp.int32),
                  jax.ShapeDtypeStruct((EP,), jnp.int32)),
        mesh=_mesh(),
        compiler_params=pltpu.CompilerParams(needs_layout_passes=False, use_tc_tiling_on_sc=False),
        scratch_types=[
            pltpu.VMEM((FC,), jnp.int32),
            pltpu.VMEM((FC,), jnp.int32),
            pltpu.VMEM((RING,), jnp.int32),
            pltpu.VMEM((RING,), jnp.int32),
            pltpu.VMEM((48,), jnp.int32),
        ],
    )
    return f(src, dst, starts48)


def _tables(h, W1, b1):
    F = h.shape[1]
    BN = 10000

    def body(x_ref, w_ref, b_ref, a_ref, bb_ref):
        xb = x_ref[...]
        w = w_ref[...]
        wa = w[:F, :]
        wb = w[F:, :]
        bb_ref[...] = jnp.dot(xb, wb, preferred_element_type=jnp.float32)
        a_ref[...] = (jnp.dot(xb, wa - wb, preferred_element_type=jnp.float32)
                      + b_ref[...])

    return pl.pallas_call(
        body,
        grid=(N // BN,),
        in_specs=[
            pl.BlockSpec((BN, F), lambda i: (i, 0)),
            pl.BlockSpec((2 * F, H), lambda i: (0, 0)),
            pl.BlockSpec((1, H), lambda i: (0, 0)),
        ],
        out_specs=[
            pl.BlockSpec((BN, H), lambda i: (i, 0)),
            pl.BlockSpec((BN, H), lambda i: (i, 0)),
        ],
        out_shape=[jax.ShapeDtypeStruct((N, H), jnp.float32)] * 2,
    )(h, W1, b1.reshape(1, H))


def _phase1(A, B, bsrc, bdst, starts48):
    def body(a_hbm, b_hbm, bsrc_hbm, bdst_hbm, starts_hbm, p_hbm,
             stv, dbuf, sbuf, arows, brows, pbuf, sema, semb):
        wid = _wid()
        pltpu.sync_copy(starts_hbm, stv)
        sv = stv[pl.ds(wid, 16)]
        st = sv[0]
        nch = (sv[1] - st) >> 7
        nmax = jnp.full((16,), N - 1, jnp.int32)
        zf = jnp.zeros((16,), jnp.float32)

        def chunk(ci, _):
            off = pl.multiple_of(st + ci * GC, GC)
            pltpu.sync_copy(bdst_hbm.at[pl.ds(off, GC)], dbuf)
            pltpu.sync_copy(bsrc_hbm.at[pl.ds(off, GC)], sbuf)
            for v in range(GC // 16):
                sl = pl.ds(v * 16, 16)
                dbuf[sl] = jnp.minimum(dbuf[sl], nmax)
            ca = pltpu.async_copy(a_hbm.at[dbuf], arows, sema)
            cb = pltpu.async_copy(b_hbm.at[sbuf], brows, semb)
            ca.wait()
            cb.wait()

            def row(r, _):
                for hh in range(2):
                    sl = pl.ds(hh * 16, 16)
                    pbuf[r, sl] = jnp.maximum(arows[r, sl] + brows[r, sl], zf)
                return 0

            lax.fori_loop(0, GC, row, 0)
            pltpu.sync_copy(pbuf, p_hbm.at[pl.ds(off, GC)])
            return 0

        lax.fori_loop(0, nch, chunk, 0)

    f = pl.kernel(
        body,
        out_type=jax.ShapeDtypeStruct((EP, H), jnp.float32),
        mesh=_mesh(),
        compiler_params=pltpu.CompilerParams(needs_layout_passes=False, use_tc_tiling_on_sc=False),
        scratch_types=[
            pltpu.VMEM((48,), jnp.int32),
            pltpu.VMEM((GC,), jnp.int32),
            pltpu.VMEM((GC,), jnp.int32),
            pltpu.VMEM((GC, H), jnp.float32),
            pltpu.VMEM((GC, H), jnp.float32),
            pltpu.VMEM((GC, H), jnp.float32),
            pltpu.SemaphoreType.DMA,
            pltpu.SemaphoreType.DMA,
        ],
    )
    return f(A, B, bsrc, bdst, starts48)


def _edge_mlp(P, W2):
    def body(p_ref, w_ref, q_ref):
        q_ref[...] = jnp.dot(p_ref[...], w_ref[...],
                             preferred_element_type=jnp.float32)

    return pl.pallas_call(
        body,
        grid=(EP // BM,),
        in_specs=[
            pl.BlockSpec((BM, H), lambda i: (i, 0)),
            pl.BlockSpec((H, H), lambda i: (0, 0)),
        ],
        out_specs=pl.BlockSpec((BM, H), lambda i: (i, 0)),
        out_shape=jax.ShapeDtypeStruct((EP, H), jnp.float32),
    )(P, W2)


def _phase2(Q, bdst, starts48, b2, relu):
    def body(q_hbm, bdst_hbm, starts_hbm, b2_hbm, h_hbm,
             stv, b2v, dbuf, qbuf, accum):
        wid = _wid()
        base = wid * NPT
        pltpu.sync_copy(starts_hbm, stv)
        pltpu.sync_copy(b2_hbm, b2v)
        sv = stv[pl.ds(wid, 16)]
        st = sv[0]
        nch = (sv[1] - st) >> 7
        neg = jnp.full((16,), NEG, jnp.float32)
        basev = jnp.full((16,), base, jnp.int32)
        zf = jnp.zeros((16,), jnp.float32)

        def init(i, _):
            accum[i, pl.ds(0, 16)] = neg
            accum[i, pl.ds(16, 16)] = neg
            return 0

        lax.fori_loop(0, NPT + 1, init, 0)

        def chunk(ci, _):
            off = pl.multiple_of(st + ci * GC, GC)
            pltpu.sync_copy(q_hbm.at[pl.ds(off, GC)], qbuf)
            pltpu.sync_copy(bdst_hbm.at[pl.ds(off, GC)], dbuf.at[pl.ds(0, GC)])
            for v in range(GC // 16):
                sl = pl.ds(v * 16, 16)
                dbuf[sl] = dbuf[sl] - basev

            def edge(e, _):
                dl = dbuf[pl.ds(e, 16)][0]
                for hh in range(2):
                    sl = pl.ds(hh * 16, 16)
                    accum[dl, sl] = jnp.maximum(accum[dl, sl], qbuf[e, sl])
                return 0

            lax.fori_loop(0, GC, edge, 0)
            return 0

        lax.fori_loop(0, nch, chunk, 0)

        def post(i, _):
            for hh in range(2):
                sl = pl.ds(hh * 16, 16)
                v = accum[i, sl]
                m = v > jnp.full((16,), THRESH, jnp.float32)
                r = jnp.where(m, v + b2v[sl], zf)
                if relu:
                    r = jnp.maximum(r, zf)
                accum[i, sl] = r
            return 0

        lax.fori_loop(0, NPT, post, 0)
        pltpu.sync_copy(accum.at[pl.ds(0, NPT)], h_hbm.at[pl.ds(base, NPT)])

    f = pl.kernel(
        body,
        out_type=jax.ShapeDtypeStruct((N, H), jnp.float32),
        mesh=_mesh(),
        compiler_params=pltpu.CompilerParams(needs_layout_passes=False, use_tc_tiling_on_sc=False),
        scratch_types=[
            pltpu.VMEM((48,), jnp.int32),
            pltpu.VMEM((H,), jnp.float32),
            pltpu.VMEM((GC + 16,), jnp.int32),
            pltpu.VMEM((GC, H), jnp.float32),
            pltpu.VMEM((NPT + 8, H), jnp.float32),
        ],
    )
    return f(Q, bdst, starts48, b2)


def _pool_head(batch, h, pW1, pb1, pW2, pb2, hW1, hb1, hW2, hb2):
    BN = 10000
    GRID = N // BN
    G = 64

    def body(b_ref, h_ref, pw1, pb1r, pw2, pb2r, hw1, hb1r, hw2, hb2r,
             out_ref, sums, cnts):
        i = pl.program_id(0)

        @pl.when(i == 0)
        def _():
            sums[...] = jnp.zeros_like(sums)
            cnts[...] = jnp.zeros_like(cnts)

        b = b_ref[0]
        oh = (b == lax.broadcasted_iota(jnp.int32, (1, G), 1)
              ).astype(jnp.float32)
        hb = h_ref[...]
        sums[...] += lax.dot_general(oh, hb, (((0,), (0,)), ((), ())),
                                     preferred_element_type=jnp.float32)
        cnts[...] += jnp.sum(oh, axis=0, keepdims=True)

        @pl.when(i == GRID - 1)
        def _():
            cnt = jnp.maximum(cnts[...], 1.0)
            mean = sums[...] / cnt.reshape(G, 1)
            z = jnp.maximum(
                jnp.dot(mean, pw1[...], preferred_element_type=jnp.float32)
                + pb1r[...], 0.0)
            z = (jnp.dot(z, pw2[...], preferred_element_type=jnp.float32)
                 + pb2r[...])
            zp = jnp.maximum(
                jnp.dot(z, hw1[...], preferred_element_type=jnp.float32)
                + hb1r[...], 0.0)
            out_ref[...] = (
                jnp.dot(zp, hw2[...], preferred_element_type=jnp.float32)
                + hb2r[...])

    batch3 = batch.reshape(GRID, BN, 1)
    full = lambda shape: pl.BlockSpec(shape, lambda i: tuple(0 for _ in shape))
    return pl.pallas_call(
        body,
        grid=(GRID,),
        in_specs=[
            pl.BlockSpec((1, BN, 1), lambda i: (i, 0, 0)),
            pl.BlockSpec((BN, H), lambda i: (i, 0)),
            full((H, 2 * H)), full((1, 2 * H)),
            full((2 * H, 64)), full((1, 64)),
            full((64, 64)), full((1, 64)),
            full((64, 32)), full((1, 32)),
        ],
        out_specs=pl.BlockSpec((G, 32), lambda i: (0, 0)),
        out_shape=jax.ShapeDtypeStruct((G, 32), jnp.float32),
        scratch_shapes=[
            pltpu.VMEM((G, H), jnp.float32),
            pltpu.VMEM((1, G), jnp.float32),
        ],
    )(batch3, h, pW1, pb1.reshape(1, -1), pW2, pb2.reshape(1, -1),
      hW1, hb1.reshape(1, -1), hW2, hb2.reshape(1, -1))


def kernel(x, edge_index, batch,
           c1_W1, c1_b1, c1_W2, c1_b2,
           c2_W1, c2_b1, c2_W2, c2_b2,
           c3_W1, c3_b1, c3_W2, c3_b2,
           proj_W1, proj_b1, proj_W2, proj_b2,
           head_W1, head_b1, head_W2, head_b2):
    src = edge_index[0]
    dst = edge_index[1]
    counts = _count_edges(dst).sum(axis=1).astype(jnp.int32)
    caps = ((counts + (QUANT - 1)) // QUANT) * QUANT
    starts = jnp.concatenate(
        [jnp.zeros((1,), jnp.int32), jnp.cumsum(caps)]).astype(jnp.int32)
    starts48 = jnp.pad(starts, (0, 48 - starts.shape[0]))
    bsrc, bdst = _bucket_edges(src, dst, starts48)

    h = x
    for (W1, b1, W2, b2, relu) in (
            (c1_W1, c1_b1, c1_W2, c1_b2, True),
            (c2_W1, c2_b1, c2_W2, c2_b2, True),
            (c3_W1, c3_b1, c3_W2, c3_b2, False)):
        A, B = _tables(h, W1, b1)
        P = _phase1(A, B, bsrc, bdst, starts48)
        Q = _edge_mlp(P, W2)
        h = _phase2(Q, bdst, starts48, b2, relu)

    return _pool_head(batch, h,
                      proj_W1, proj_b1, proj_W2, proj_b2,
                      head_W1, head_b1, head_W2, head_b2)


# speedup vs baseline: 2.6762x; 1.3388x over previous
import functools

import jax
import jax.numpy as jnp
from jax import lax
from jax.experimental import pallas as pl
from jax.experimental.pallas import tpu as pltpu
from jax.experimental.pallas import tpu_sc as plsc

N = 100000
E = 1600000
H = 32
NT = 32
NPT = N // NT
FC = 4000
QUANT = 1024
RING = 8192
GC = 128
EP = E + NT * QUANT
BM = 1536
NEG = -3.0e38
THRESH = -1.0e38

_mesh = functools.partial(
    plsc.VectorSubcoreMesh, core_axis_name="c", subcore_axis_name="s")


def _wid():
    return lax.axis_index("s") * 2 + lax.axis_index("c")


def _count_edges(dst):
    EPT = E // NT
    CH = 2000
    NCH = EPT // CH

    def body(dst_hbm, cnt_hbm, dbuf0, dbuf1, hist, sem0, sem1):
        wid = _wid()
        ebase = wid * EPT
        bufs = (dbuf0, dbuf1)
        sems = (sem0, sem1)
        for b in range(NT):
            hist[pl.ds(b * 16, 16)] = jnp.zeros((16,), jnp.int32)
        pltpu.async_copy(
            dst_hbm.at[pl.ds(pl.multiple_of(ebase, 8), CH)], dbuf0, sem0)

        def process(buf):
            def vec(vi, _):
                d = buf[pl.ds(vi * 16, 16)]
                for b in range(NT):
                    m = d >= jnp.full((16,), b * NPT, jnp.int32)
                    sl = pl.ds(b * 16, 16)
                    hist[sl] = hist[sl] + jnp.where(m, 1, 0)
                return 0

            lax.fori_loop(0, CH // 16, vec, 0)

        def step(ci, _):
            for par in range(2):
                @pl.when((ci & 1) == par)
                def _():
                    pltpu.make_async_copy(
                        dst_hbm.at[pl.ds(0, CH)], bufs[par], sems[par]).wait()

                    @pl.when(ci + 1 < NCH)
                    def _():
                        off = pl.multiple_of(ebase + (ci + 1) * CH, 8)
                        pltpu.async_copy(dst_hbm.at[pl.ds(off, CH)],
                                         bufs[1 - par], sems[1 - par])

                    process(bufs[par])
            return 0

        lax.fori_loop(0, NCH, step, 0)
        pltpu.sync_copy(hist, cnt_hbm.at[wid])

    f = pl.kernel(
        body,
        out_type=jax.ShapeDtypeStruct((NT, NT * 16), jnp.int32),
        mesh=_mesh(),
        compiler_params=pltpu.CompilerParams(needs_layout_passes=False, use_tc_tiling_on_sc=False),
        scratch_types=[
            pltpu.VMEM((CH,), jnp.int32),
            pltpu.VMEM((CH,), jnp.int32),
            pltpu.VMEM((NT * 16,), jnp.int32),
            pltpu.SemaphoreType.DMA,
            pltpu.SemaphoreType.DMA,
        ],
    )
    return f(dst)


def _bucket_edges(src, dst, starts48):
    def body(src_hbm, dst_hbm, starts_hbm, bsrc_hbm, bdst_hbm,
             sbuf, dbuf, rings, ringd, stv):
        wid = _wid()
        base = wid * NPT
        lo = jnp.full((16,), base, jnp.int32)
        hi = jnp.full((16,), base + NPT, jnp.int32)
        sent = jnp.full((16,), base + NPT, jnp.int32)
        zero16 = jnp.zeros((16,), jnp.int32)
        lane = lax.iota(jnp.int32, 16)
        pltpu.sync_copy(starts_hbm, stv)
        st = stv[pl.ds(wid, 16)][0]

        def flush_while(cur, flushed):
            def cond(f):
                return cur - f >= QUANT

            def fbody(f):
                off = pl.multiple_of(f & (RING - 1), QUANT)
                dsto = pl.multiple_of(st + f, QUANT)
                pltpu.sync_copy(ringd.at[pl.ds(off, QUANT)],
                                bdst_hbm.at[pl.ds(dsto, QUANT)])
                pltpu.sync_copy(rings.at[pl.ds(off, QUANT)],
                                bsrc_hbm.at[pl.ds(dsto, QUANT)])
                return f + QUANT

            return lax.while_loop(cond, fbody, flushed)

        def chunk(ci, carry):
            curv, flushed = carry
            co = pl.multiple_of(ci * FC, FC)
            pltpu.sync_copy(src_hbm.at[pl.ds(co, FC)], sbuf)
            pltpu.sync_copy(dst_hbm.at[pl.ds(co, FC)], dbuf)

            def vec(vi, cv):
                d = dbuf[pl.ds(vi * 16, 16)]
                s = sbuf[pl.ds(vi * 16, 16)]
                m = (d >= lo) & (d < hi)
                csum = plsc.cumsum(jnp.where(m, 1, 0))
                pos = (cv + csum - 1) & (RING - 1)
                plsc.store_scatter(ringd, [pos], d, mask=m)
                plsc.store_scatter(rings, [pos], s, mask=m)
                return cv + csum[15]

            curv = lax.fori_loop(0, FC // 16, vec, curv)
            flushed = flush_while(curv[0], flushed)
            return curv, flushed

        curv, flushed = lax.fori_loop(
            0, E // FC, chunk, (jnp.zeros((16,), jnp.int32), jnp.int32(0)))

        tgt = ((curv + (QUANT - 1)) >> 10) << 10
        for j in range(QUANT // 16):
            pos = curv + j * 16 + lane
            m = pos < tgt
            plsc.store_scatter(ringd, [pos & (RING - 1)], sent, mask=m)
            plsc.store_scatter(rings, [pos & (RING - 1)], zero16, mask=m)
        flushed = flush_while(tgt[0], flushed)

    f = pl.kernel(
        body,
        out_type=(jax.ShapeDtypeStruct((EP,), jnp.int32),
                  jax.ShapeDtypeStruct((EP,), jnp.int32)),
        mesh=_mesh(),
        compiler_params=pltpu.CompilerParams(needs_layout_passes=False, use_tc_tiling_on_sc=False),
        scratch_types=[
            pltpu.VMEM((FC,), jnp.int32),
            pltpu.VMEM((FC,), jnp.int32),
            pltpu.VMEM((RING,), jnp.int32),
            pltpu.VMEM((RING,), jnp.int32),
            pltpu.VMEM((48,), jnp.int32),
        ],
    )
    return f(src, dst, starts48)


def _tables(h, W1, b1):
    F = h.shape[1]
    BN = 10000

    def body(x_ref, w_ref, b_ref, a_ref, bb_ref):
        xb = x_ref[...]
        w = w_ref[...]
        wa = w[:F, :]
        wb = w[F:, :]
        bb_ref[...] = jnp.dot(xb, wb, preferred_element_type=jnp.float32)
        a_ref[...] = (jnp.dot(xb, wa - wb, preferred_element_type=jnp.float32)
                      + b_ref[...])

    return pl.pallas_call(
        body,
        grid=(N // BN,),
        in_specs=[
            pl.BlockSpec((BN, F), lambda i: (i, 0)),
            pl.BlockSpec((2 * F, H), lambda i: (0, 0)),
            pl.BlockSpec((1, H), lambda i: (0, 0)),
        ],
        out_specs=[
            pl.BlockSpec((BN, H), lambda i: (i, 0)),
            pl.BlockSpec((BN, H), lambda i: (i, 0)),
        ],
        out_shape=[jax.ShapeDtypeStruct((N, H), jnp.float32)] * 2,
    )(h, W1, b1.reshape(1, H))


SUP = 512
NG = SUP // GC


def _phase1(A, B, bsrc, bdst, starts48):
    def body(a_hbm, b_hbm, bsrc_hbm, bdst_hbm, starts_hbm, p_hbm,
             stv, d0, s0, d1, s1, ar0, br0, ar1, br1, pb0, pb1,
             sg0, sg1, si0, si1, pw0, pw1):
        wid = _wid()
        pltpu.sync_copy(starts_hbm, stv)
        sv = stv[pl.ds(wid, 16)]
        st = sv[0]
        nch = (sv[1] - st) >> 9
        nmax = jnp.full((16,), N - 1, jnp.int32)
        zf = jnp.zeros((16,), jnp.float32)
        ibd = (d0, d1)
        ibs = (s0, s1)
        ars = (ar0, ar1)
        brs = (br0, br1)
        pbs = (pb0, pb1)
        sg = (sg0, sg1)
        si = (si0, si1)
        pw = (pw0, pw1)

        def off_of(ci):
            return pl.multiple_of(st + ci * SUP, GC)

        def issue_idx(ci, b):
            off = off_of(ci)
            pltpu.async_copy(bdst_hbm.at[pl.ds(off, SUP)], ibd[b], si[b])
            pltpu.async_copy(bsrc_hbm.at[pl.ds(off, SUP)], ibs[b], si[b])

        def wait_idx(b):
            pltpu.make_async_copy(
                bdst_hbm.at[pl.ds(0, SUP)], ibd[b], si[b]).wait()
            pltpu.make_async_copy(
                bsrc_hbm.at[pl.ds(0, SUP)], ibs[b], si[b]).wait()

        def issue_gathers(b):
            for v in range(SUP // 16):
                sl = pl.ds(v * 16, 16)
                ibd[b][sl] = jnp.minimum(ibd[b][sl], nmax)
            for g in range(NG):
                gs = pl.ds(g * GC, GC)
                pltpu.async_copy(a_hbm.at[ibd[b].at[gs]], ars[b].at[gs], sg[b])
                pltpu.async_copy(b_hbm.at[ibs[b].at[gs]], brs[b].at[gs], sg[b])

        def wait_gathers(b):
            for g in range(2 * NG):
                pltpu.make_async_copy(
                    p_hbm.at[pl.ds(0, GC)], ars[b].at[pl.ds(0, GC)],
                    sg[b]).wait()

        def compute(ci, b):
            def row(r0, _):
                for u in range(4):
                    r = r0 * 4 + u
                    for hh in range(2):
                        sl = pl.ds(hh * 16, 16)
                        pbs[b][r, sl] = jnp.maximum(
                            ars[b][r, sl] + brs[b][r, sl], zf)
                return 0

            lax.fori_loop(0, SUP // 4, row, 0)
            pltpu.async_copy(pbs[b], p_hbm.at[pl.ds(off_of(ci), SUP)], pw[b])

        def wait_pw(b):
            pltpu.make_async_copy(
                p_hbm.at[pl.ds(0, SUP)], pbs[b], pw[b]).wait()

        @pl.when(nch > 0)
        def _():
            issue_idx(0, 0)
            wait_idx(0)
            issue_gathers(0)
            issue_idx(1, 1)

            def step(ci, _):
                for par in range(2):
                    @pl.when((ci & 1) == par)
                    def _():
                        wait_gathers(par)

                        @pl.when(ci + 1 < nch)
                        def _():
                            wait_idx(1 - par)
                            issue_gathers(1 - par)

                        @pl.when(ci + 2 < nch)
                        def _():
                            issue_idx(ci + 2, par)

                        @pl.when(ci >= 2)
                        def _():
                            wait_pw(par)

                        compute(ci, par)
                return 0

            lax.fori_loop(0, nch, step, 0)
            wait_pw(0)
            wait_pw(1)

    f = pl.kernel(
        body,
        out_type=jax.ShapeDtypeStruct((EP, H), jnp.float32),
        mesh=_mesh(),
        compiler_params=pltpu.CompilerParams(needs_layout_passes=False, use_tc_tiling_on_sc=False),
        scratch_types=[
            pltpu.VMEM((48,), jnp.int32),
            pltpu.VMEM((SUP,), jnp.int32),
            pltpu.VMEM((SUP,), jnp.int32),
            pltpu.VMEM((SUP,), jnp.int32),
            pltpu.VMEM((SUP,), jnp.int32),
            pltpu.VMEM((SUP, H), jnp.float32),
            pltpu.VMEM((SUP, H), jnp.float32),
            pltpu.VMEM((SUP, H), jnp.float32),
            pltpu.VMEM((SUP, H), jnp.float32),
            pltpu.VMEM((SUP, H), jnp.float32),
            pltpu.VMEM((SUP, H), jnp.float32),
            pltpu.SemaphoreType.DMA,
            pltpu.SemaphoreType.DMA,
            pltpu.SemaphoreType.DMA,
            pltpu.SemaphoreType.DMA,
            pltpu.SemaphoreType.DMA,
            pltpu.SemaphoreType.DMA,
        ],
    )
    return f(A, B, bsrc, bdst, starts48)


def _edge_mlp(P, W2):
    def body(p_ref, w_ref, q_ref):
        q_ref[...] = jnp.dot(p_ref[...], w_ref[...],
                             preferred_element_type=jnp.float32)

    return pl.pallas_call(
        body,
        grid=(EP // BM,),
        in_specs=[
            pl.BlockSpec((BM, H), lambda i: (i, 0)),
            pl.BlockSpec((H, H), lambda i: (0, 0)),
        ],
        out_specs=pl.BlockSpec((BM, H), lambda i: (i, 0)),
        out_shape=jax.ShapeDtypeStruct((EP, H), jnp.float32),
    )(P, W2)


def _phase2(Q, bdst, starts48, b2, relu):
    QC = 256

    def body(q_hbm, bdst_hbm, starts_hbm, b2_hbm, h_hbm,
             stv, b2v, db0, db1, qb0, qb1, accum, sq0, sq1):
        wid = _wid()
        base = wid * NPT
        pltpu.sync_copy(starts_hbm, stv)
        pltpu.sync_copy(b2_hbm, b2v)
        sv = stv[pl.ds(wid, 16)]
        st = sv[0]
        nch = (sv[1] - st) >> 8
        neg = jnp.full((16,), NEG, jnp.float32)
        basev = jnp.full((16,), base, jnp.int32)
        zf = jnp.zeros((16,), jnp.float32)
        dbs = (db0, db1)
        qbs = (qb0, qb1)
        sq = (sq0, sq1)

        def init(i, _):
            for u in range(4):
                accum[i * 4 + u, pl.ds(0, 16)] = neg
                accum[i * 4 + u, pl.ds(16, 16)] = neg
            return 0

        lax.fori_loop(0, (NPT + 4) // 4, init, 0)

        def issue(ci, b):
            off = pl.multiple_of(st + ci * QC, QC)
            pltpu.async_copy(q_hbm.at[pl.ds(off, QC)], qbs[b], sq[b])
            pltpu.async_copy(bdst_hbm.at[pl.ds(off, QC)],
                             dbs[b].at[pl.ds(0, QC)], sq[b])

        def wait_io(b):
            pltpu.make_async_copy(q_hbm.at[pl.ds(0, QC)], qbs[b],
                                  sq[b]).wait()
            pltpu.make_async_copy(bdst_hbm.at[pl.ds(0, QC)],
                                  dbs[b].at[pl.ds(0, QC)], sq[b]).wait()

        def process(b):
            dbuf = dbs[b]
            qbuf = qbs[b]
            for v in range(QC // 16):
                sl = pl.ds(v * 16, 16)
                dbuf[sl] = dbuf[sl] - basev

            def edge(e0, _):
                for u in range(4):
                    e = e0 * 4 + u
                    dl = dbuf[pl.ds(e, 16)][0]
                    for hh in range(2):
                        sl = pl.ds(hh * 16, 16)
                        accum[dl, sl] = jnp.maximum(accum[dl, sl],
                                                    qbuf[e, sl])
                return 0

            lax.fori_loop(0, QC // 4, edge, 0)

        @pl.when(nch > 0)
        def _():
            issue(0, 0)

            def step(ci, _):
                for par in range(2):
                    @pl.when((ci & 1) == par)
                    def _():
                        wait_io(par)

                        @pl.when(ci + 1 < nch)
                        def _():
                            issue(ci + 1, 1 - par)

                        process(par)
                return 0

            lax.fori_loop(0, nch, step, 0)

        def post(i, _):
            for hh in range(2):
                sl = pl.ds(hh * 16, 16)
                v = accum[i, sl]
                m = v > jnp.full((16,), THRESH, jnp.float32)
                r = jnp.where(m, v + b2v[sl], zf)
                if relu:
                    r = jnp.maximum(r, zf)
                accum[i, sl] = r
            return 0

        lax.fori_loop(0, NPT, post, 0)
        pltpu.sync_copy(accum.at[pl.ds(0, NPT)], h_hbm.at[pl.ds(base, NPT)])

    f = pl.kernel(
        body,
        out_type=jax.ShapeDtypeStruct((N, H), jnp.float32),
        mesh=_mesh(),
        compiler_params=pltpu.CompilerParams(needs_layout_passes=False, use_tc_tiling_on_sc=False),
        scratch_types=[
            pltpu.VMEM((48,), jnp.int32),
            pltpu.VMEM((H,), jnp.float32),
            pltpu.VMEM((QC + 16,), jnp.int32),
            pltpu.VMEM((QC + 16,), jnp.int32),
            pltpu.VMEM((QC, H), jnp.float32),
            pltpu.VMEM((QC, H), jnp.float32),
            pltpu.VMEM((NPT + 8, H), jnp.float32),
            pltpu.SemaphoreType.DMA,
            pltpu.SemaphoreType.DMA,
        ],
    )
    return f(Q, bdst, starts48, b2)


def _pool_head(batch, h, pW1, pb1, pW2, pb2, hW1, hb1, hW2, hb2):
    BN = 10000
    GRID = N // BN
    G = 64

    def body(b_ref, h_ref, pw1, pb1r, pw2, pb2r, hw1, hb1r, hw2, hb2r,
             out_ref, sums, cnts):
        i = pl.program_id(0)

        @pl.when(i == 0)
        def _():
            sums[...] = jnp.zeros_like(sums)
            cnts[...] = jnp.zeros_like(cnts)

        b = b_ref[0]
        oh = (b == lax.broadcasted_iota(jnp.int32, (1, G), 1)
              ).astype(jnp.float32)
        hb = h_ref[...]
        sums[...] += lax.dot_general(oh, hb, (((0,), (0,)), ((), ())),
                                     preferred_element_type=jnp.float32)
        cnts[...] += jnp.sum(oh, axis=0, keepdims=True)

        @pl.when(i == GRID - 1)
        def _():
            cnt = jnp.maximum(cnts[...], 1.0)
            mean = sums[...] / cnt.reshape(G, 1)
            z = jnp.maximum(
                jnp.dot(mean, pw1[...], preferred_element_type=jnp.float32)
                + pb1r[...], 0.0)
            z = (jnp.dot(z, pw2[...], preferred_element_type=jnp.float32)
                 + pb2r[...])
            zp = jnp.maximum(
                jnp.dot(z, hw1[...], preferred_element_type=jnp.float32)
                + hb1r[...], 0.0)
            out_ref[...] = (
                jnp.dot(zp, hw2[...], preferred_element_type=jnp.float32)
                + hb2r[...])

    batch3 = batch.reshape(GRID, BN, 1)
    full = lambda shape: pl.BlockSpec(shape, lambda i: tuple(0 for _ in shape))
    return pl.pallas_call(
        body,
        grid=(GRID,),
        in_specs=[
            pl.BlockSpec((1, BN, 1), lambda i: (i, 0, 0)),
            pl.BlockSpec((BN, H), lambda i: (i, 0)),
            full((H, 2 * H)), full((1, 2 * H)),
            full((2 * H, 64)), full((1, 64)),
            full((64, 64)), full((1, 64)),
            full((64, 32)), full((1, 32)),
        ],
        out_specs=pl.BlockSpec((G, 32), lambda i: (0, 0)),
        out_shape=jax.ShapeDtypeStruct((G, 32), jnp.float32),
        scratch_shapes=[
            pltpu.VMEM((G, H), jnp.float32),
            pltpu.VMEM((1, G), jnp.float32),
        ],
    )(batch3, h, pW1, pb1.reshape(1, -1), pW2, pb2.reshape(1, -1),
      hW1, hb1.reshape(1, -1), hW2, hb2.reshape(1, -1))


def kernel(x, edge_index, batch,
           c1_W1, c1_b1, c1_W2, c1_b2,
           c2_W1, c2_b1, c2_W2, c2_b2,
           c3_W1, c3_b1, c3_W2, c3_b2,
           proj_W1, proj_b1, proj_W2, proj_b2,
           head_W1, head_b1, head_W2, head_b2):
    src = edge_index[0]
    dst = edge_index[1]
    ge_counts = _count_edges(dst).reshape(NT, NT, 16).sum(
        axis=(0, 2)).astype(jnp.int32)
    counts = ge_counts - jnp.concatenate(
        [ge_counts[1:], jnp.zeros((1,), jnp.int32)])
    caps = ((counts + (QUANT - 1)) // QUANT) * QUANT
    starts = jnp.concatenate(
        [jnp.zeros((1,), jnp.int32), jnp.cumsum(caps)]).astype(jnp.int32)
    starts48 = jnp.pad(starts, (0, 48 - starts.shape[0]))
    bsrc, bdst = _bucket_edges(src, dst, starts48)

    h = x
    for (W1, b1, W2, b2, relu) in (
            (c1_W1, c1_b1, c1_W2, c1_b2, True),
            (c2_W1, c2_b1, c2_W2, c2_b2, True),
            (c3_W1, c3_b1, c3_W2, c3_b2, False)):
        A, B = _tables(h, W1, b1)
        P = _phase1(A, B, bsrc, bdst, starts48)
        Q = _edge_mlp(P, W2)
        h = _phase2(Q, bdst, starts48, b2, relu)

    return _pool_head(batch, h,
                      proj_W1, proj_b1, proj_W2, proj_b2,
                      head_W1, head_b1, head_W2, head_b2)


# speedup vs baseline: 3.8818x; 1.4505x over previous
import functools

import jax
import jax.numpy as jnp
from jax import lax
from jax.experimental import pallas as pl
from jax.experimental.pallas import tpu as pltpu
from jax.experimental.pallas import tpu_sc as plsc

N = 100000
E = 1600000
H = 32
NT = 32
NPT = N // NT
FC = 4000
QUANT = 1024
RING = 8192
GC = 128
EP = E + NT * QUANT
BM = 1536
NEG = -3.0e38
THRESH = -1.0e38

_mesh = functools.partial(
    plsc.VectorSubcoreMesh, core_axis_name="c", subcore_axis_name="s")


def _wid():
    return lax.axis_index("s") * 2 + lax.axis_index("c")


def _count_edges(dst):
    EPT = E // NT
    CH = 2000
    NCH = EPT // CH

    def body(dst_hbm, cnt_hbm, dbuf0, dbuf1, hist, sem0, sem1):
        wid = _wid()
        ebase = wid * EPT
        bufs = (dbuf0, dbuf1)
        sems = (sem0, sem1)
        for b in range(NT):
            hist[pl.ds(b * 16, 16)] = jnp.zeros((16,), jnp.int32)
        pltpu.async_copy(
            dst_hbm.at[pl.ds(pl.multiple_of(ebase, 8), CH)], dbuf0, sem0)

        def process(buf):
            def vec(vi, _):
                d = buf[pl.ds(vi * 16, 16)]
                for b in range(NT):
                    m = d >= jnp.full((16,), b * NPT, jnp.int32)
                    sl = pl.ds(b * 16, 16)
                    hist[sl] = hist[sl] + jnp.where(m, 1, 0)
                return 0

            lax.fori_loop(0, CH // 16, vec, 0)

        def step(ci, _):
            for par in range(2):
                @pl.when((ci & 1) == par)
                def _():
                    pltpu.make_async_copy(
                        dst_hbm.at[pl.ds(0, CH)], bufs[par], sems[par]).wait()

                    @pl.when(ci + 1 < NCH)
                    def _():
                        off = pl.multiple_of(ebase + (ci + 1) * CH, 8)
                        pltpu.async_copy(dst_hbm.at[pl.ds(off, CH)],
                                         bufs[1 - par], sems[1 - par])

                    process(bufs[par])
            return 0

        lax.fori_loop(0, NCH, step, 0)
        pltpu.sync_copy(hist, cnt_hbm.at[wid])

    f = pl.kernel(
        body,
        out_type=jax.ShapeDtypeStruct((NT, NT * 16), jnp.int32),
        mesh=_mesh(),
        compiler_params=pltpu.CompilerParams(needs_layout_passes=False, use_tc_tiling_on_sc=False),
        scratch_types=[
            pltpu.VMEM((CH,), jnp.int32),
            pltpu.VMEM((CH,), jnp.int32),
            pltpu.VMEM((NT * 16,), jnp.int32),
            pltpu.SemaphoreType.DMA,
            pltpu.SemaphoreType.DMA,
        ],
    )
    return f(dst)


def _bucket_edges(src, dst, starts48):
    def body(src_hbm, dst_hbm, starts_hbm, bsrc_hbm, bdst_hbm,
             sbuf, dbuf, rings, ringd, stv):
        wid = _wid()
        base = wid * NPT
        lo = jnp.full((16,), base, jnp.int32)
        hi = jnp.full((16,), base + NPT, jnp.int32)
        sent = jnp.full((16,), base + NPT, jnp.int32)
        zero16 = jnp.zeros((16,), jnp.int32)
        lane = lax.iota(jnp.int32, 16)
        pltpu.sync_copy(starts_hbm, stv)
        st = stv[pl.ds(wid, 16)][0]

        def flush_while(cur, flushed):
            def cond(f):
                return cur - f >= QUANT

            def fbody(f):
                off = pl.multiple_of(f & (RING - 1), QUANT)
                dsto = pl.multiple_of(st + f, QUANT)
                pltpu.sync_copy(ringd.at[pl.ds(off, QUANT)],
                                bdst_hbm.at[pl.ds(dsto, QUANT)])
                pltpu.sync_copy(rings.at[pl.ds(off, QUANT)],
                                bsrc_hbm.at[pl.ds(dsto, QUANT)])
                return f + QUANT

            return lax.while_loop(cond, fbody, flushed)

        def chunk(ci, carry):
            curv, flushed = carry
            co = pl.multiple_of(ci * FC, FC)
            pltpu.sync_copy(src_hbm.at[pl.ds(co, FC)], sbuf)
            pltpu.sync_copy(dst_hbm.at[pl.ds(co, FC)], dbuf)

            def vec(vi, cv):
                d = dbuf[pl.ds(vi * 16, 16)]
                s = sbuf[pl.ds(vi * 16, 16)]
                m = (d >= lo) & (d < hi)
                csum = plsc.cumsum(jnp.where(m, 1, 0))
                pos = (cv + csum - 1) & (RING - 1)
                plsc.store_scatter(ringd, [pos], d, mask=m)
                plsc.store_scatter(rings, [pos], s, mask=m)
                return cv + csum[15]

            curv = lax.fori_loop(0, FC // 16, vec, curv)
            flushed = flush_while(curv[0], flushed)
            return curv, flushed

        curv, flushed = lax.fori_loop(
            0, E // FC, chunk, (jnp.zeros((16,), jnp.int32), jnp.int32(0)))

        tgt = ((curv + (QUANT - 1)) >> 10) << 10
        for j in range(QUANT // 16):
            pos = curv + j * 16 + lane
            m = pos < tgt
            plsc.store_scatter(ringd, [pos & (RING - 1)], sent, mask=m)
            plsc.store_scatter(rings, [pos & (RING - 1)], zero16, mask=m)
        flushed = flush_while(tgt[0], flushed)

    f = pl.kernel(
        body,
        out_type=(jax.ShapeDtypeStruct((EP,), jnp.int32),
                  jax.ShapeDtypeStruct((EP,), jnp.int32)),
        mesh=_mesh(),
        compiler_params=pltpu.CompilerParams(needs_layout_passes=False, use_tc_tiling_on_sc=False),
        scratch_types=[
            pltpu.VMEM((FC,), jnp.int32),
            pltpu.VMEM((FC,), jnp.int32),
            pltpu.VMEM((RING,), jnp.int32),
            pltpu.VMEM((RING,), jnp.int32),
            pltpu.VMEM((48,), jnp.int32),
        ],
    )
    return f(src, dst, starts48)


def _tables(h, W1, b1):
    F = h.shape[1]
    BN = 10000

    def body(x_ref, w_ref, b_ref, a_ref, bb_ref):
        xb = x_ref[...]
        w = w_ref[...]
        wa = w[:F, :]
        wb = w[F:, :]
        bb_ref[...] = jnp.dot(xb, wb, preferred_element_type=jnp.float32)
        a_ref[...] = (jnp.dot(xb, wa - wb, preferred_element_type=jnp.float32)
                      + b_ref[...])

    return pl.pallas_call(
        body,
        grid=(N // BN,),
        in_specs=[
            pl.BlockSpec((BN, F), lambda i: (i, 0)),
            pl.BlockSpec((2 * F, H), lambda i: (0, 0)),
            pl.BlockSpec((1, H), lambda i: (0, 0)),
        ],
        out_specs=[
            pl.BlockSpec((BN, H), lambda i: (i, 0)),
            pl.BlockSpec((BN, H), lambda i: (i, 0)),
        ],
        out_shape=[jax.ShapeDtypeStruct((N, H), jnp.float32)] * 2,
    )(h, W1, b1.reshape(1, H))


SUP = 512
NG = SUP // GC


def _phase1(A, B, bsrc, bdst, starts48):
    def body(a_hbm, b_hbm, bsrc_hbm, bdst_hbm, starts_hbm, p_hbm,
             stv, d0, s0, d1, s1, ar0, br0, ar1, br1, pb0, pb1,
             sg0, sg1, si0, si1, pw0, pw1):
        wid = _wid()
        pltpu.sync_copy(starts_hbm, stv)
        sv = stv[pl.ds(wid, 16)]
        st = sv[0]
        nch = (sv[1] - st) >> 9
        nmax = jnp.full((16,), N - 1, jnp.int32)
        zf = jnp.zeros((16,), jnp.float32)
        ibd = (d0, d1)
        ibs = (s0, s1)
        ars = (ar0, ar1)
        brs = (br0, br1)
        pbs = (pb0, pb1)
        sg = (sg0, sg1)
        si = (si0, si1)
        pw = (pw0, pw1)

        def off_of(ci):
            return pl.multiple_of(st + ci * SUP, GC)

        def issue_idx(ci, b):
            off = off_of(ci)
            pltpu.async_copy(bdst_hbm.at[pl.ds(off, SUP)], ibd[b], si[b])
            pltpu.async_copy(bsrc_hbm.at[pl.ds(off, SUP)], ibs[b], si[b])

        def wait_idx(b):
            pltpu.make_async_copy(
                bdst_hbm.at[pl.ds(0, SUP)], ibd[b], si[b]).wait()
            pltpu.make_async_copy(
                bsrc_hbm.at[pl.ds(0, SUP)], ibs[b], si[b]).wait()

        def issue_gathers(b):
            for v in range(SUP // 16):
                sl = pl.ds(v * 16, 16)
                ibd[b][sl] = jnp.minimum(ibd[b][sl], nmax)
            for g in range(NG):
                gs = pl.ds(g * GC, GC)
                pltpu.async_copy(a_hbm.at[ibd[b].at[gs]], ars[b].at[gs], sg[b])
                pltpu.async_copy(b_hbm.at[ibs[b].at[gs]], brs[b].at[gs], sg[b])

        def wait_gathers(b):
            for g in range(2 * NG):
                pltpu.make_async_copy(
                    p_hbm.at[pl.ds(0, GC)], ars[b].at[pl.ds(0, GC)],
                    sg[b]).wait()

        def compute(ci, b):
            def row(r0, _):
                for u in range(4):
                    r = r0 * 4 + u
                    for hh in range(2):
                        sl = pl.ds(hh * 16, 16)
                        po = pl.ds(u * 32 + hh * 16, 16)
                        pbs[b][r0, po] = jnp.maximum(
                            ars[b][r, sl] + brs[b][r, sl], zf)
                return 0

            lax.fori_loop(0, SUP // 4, row, 0)
            offq = pl.multiple_of((st + ci * SUP) >> 2, SUP // 4)
            pltpu.async_copy(pbs[b], p_hbm.at[pl.ds(offq, SUP // 4)], pw[b])

        def wait_pw(b):
            pltpu.make_async_copy(
                p_hbm.at[pl.ds(0, SUP // 4)], pbs[b], pw[b]).wait()

        @pl.when(nch > 0)
        def _():
            issue_idx(0, 0)
            wait_idx(0)
            issue_gathers(0)
            issue_idx(1, 1)

            def step(ci, _):
                for par in range(2):
                    @pl.when((ci & 1) == par)
                    def _():
                        wait_gathers(par)

                        @pl.when(ci + 1 < nch)
                        def _():
                            wait_idx(1 - par)
                            issue_gathers(1 - par)

                        @pl.when(ci + 2 < nch)
                        def _():
                            issue_idx(ci + 2, par)

                        @pl.when(ci >= 2)
                        def _():
                            wait_pw(par)

                        compute(ci, par)
                return 0

            lax.fori_loop(0, nch, step, 0)
            wait_pw(0)
            wait_pw(1)

    f = pl.kernel(
        body,
        out_type=jax.ShapeDtypeStruct((EP // 4, 4 * H), jnp.float32),
        mesh=_mesh(),
        compiler_params=pltpu.CompilerParams(needs_layout_passes=False, use_tc_tiling_on_sc=False),
        scratch_types=[
            pltpu.VMEM((48,), jnp.int32),
            pltpu.VMEM((SUP,), jnp.int32),
            pltpu.VMEM((SUP,), jnp.int32),
            pltpu.VMEM((SUP,), jnp.int32),
            pltpu.VMEM((SUP,), jnp.int32),
            pltpu.VMEM((SUP, H), jnp.float32),
            pltpu.VMEM((SUP, H), jnp.float32),
            pltpu.VMEM((SUP, H), jnp.float32),
            pltpu.VMEM((SUP, H), jnp.float32),
            pltpu.VMEM((SUP // 4, 4 * H), jnp.float32),
            pltpu.VMEM((SUP // 4, 4 * H), jnp.float32),
            pltpu.SemaphoreType.DMA,
            pltpu.SemaphoreType.DMA,
            pltpu.SemaphoreType.DMA,
            pltpu.SemaphoreType.DMA,
            pltpu.SemaphoreType.DMA,
            pltpu.SemaphoreType.DMA,
        ],
    )
    return f(A, B, bsrc, bdst, starts48)


def _edge_mlp(P, W2):
    EPQ = EP // 4
    BMQ = 384

    def body(p_ref, w_ref, q_ref):
        w = w_ref[...]
        z = jnp.zeros((H, H), jnp.float32)
        wd = jnp.concatenate(
            [jnp.concatenate([w if j == i else z for j in range(4)], axis=1)
             for i in range(4)], axis=0)
        q_ref[...] = jnp.dot(p_ref[...], wd,
                             preferred_element_type=jnp.float32)

    return pl.pallas_call(
        body,
        grid=(EPQ // BMQ,),
        in_specs=[
            pl.BlockSpec((BMQ, 4 * H), lambda i: (i, 0)),
            pl.BlockSpec((H, H), lambda i: (0, 0)),
        ],
        out_specs=pl.BlockSpec((BMQ, 4 * H), lambda i: (i, 0)),
        out_shape=jax.ShapeDtypeStruct((EPQ, 4 * H), jnp.float32),
    )(P, W2)


def _phase2(Q, bdst, starts48, b2, relu):
    QC = 256

    def body(q_hbm, bdst_hbm, starts_hbm, b2_hbm, h_hbm,
             stv, b2v, db0, db1, qb0, qb1, accum, sq0, sq1):
        wid = _wid()
        base = wid * NPT
        pltpu.sync_copy(starts_hbm, stv)
        pltpu.sync_copy(b2_hbm, b2v)
        sv = stv[pl.ds(wid, 16)]
        st = sv[0]
        nch = (sv[1] - st) >> 8
        neg = jnp.full((16,), NEG, jnp.float32)
        basev = jnp.full((16,), base, jnp.int32)
        zf = jnp.zeros((16,), jnp.float32)
        dbs = (db0, db1)
        qbs = (qb0, qb1)
        sq = (sq0, sq1)

        def init(i, _):
            for u in range(4):
                accum[i * 4 + u, pl.ds(0, 16)] = neg
                accum[i * 4 + u, pl.ds(16, 16)] = neg
            return 0

        lax.fori_loop(0, (NPT + 4) // 4, init, 0)

        def issue(ci, b):
            off = pl.multiple_of(st + ci * QC, QC)
            offq = pl.multiple_of((st + ci * QC) >> 2, QC // 4)
            pltpu.async_copy(q_hbm.at[pl.ds(offq, QC // 4)], qbs[b], sq[b])
            pltpu.async_copy(bdst_hbm.at[pl.ds(off, QC)],
                             dbs[b].at[pl.ds(0, QC)], sq[b])

        def wait_io(b):
            pltpu.make_async_copy(q_hbm.at[pl.ds(0, QC // 4)], qbs[b],
                                  sq[b]).wait()
            pltpu.make_async_copy(bdst_hbm.at[pl.ds(0, QC)],
                                  dbs[b].at[pl.ds(0, QC)], sq[b]).wait()

        def process(b):
            dbuf = dbs[b]
            qbuf = qbs[b]
            for v in range(QC // 16):
                sl = pl.ds(v * 16, 16)
                dbuf[sl] = dbuf[sl] - basev

            def edge(e0, _):
                for u in range(4):
                    e = e0 * 4 + u
                    dl = dbuf[pl.ds(e, 16)][0]
                    for hh in range(2):
                        sl = pl.ds(hh * 16, 16)
                        qo = pl.ds(u * 32 + hh * 16, 16)
                        accum[dl, sl] = jnp.maximum(accum[dl, sl],
                                                    qbuf[e0, qo])
                return 0

            lax.fori_loop(0, QC // 4, edge, 0)

        @pl.when(nch > 0)
        def _():
            issue(0, 0)

            def step(ci, _):
                for par in range(2):
                    @pl.when((ci & 1) == par)
                    def _():
                        wait_io(par)

                        @pl.when(ci + 1 < nch)
                        def _():
                            issue(ci + 1, 1 - par)

                        process(par)
                return 0

            lax.fori_loop(0, nch, step, 0)

        def post(i, _):
            for hh in range(2):
                sl = pl.ds(hh * 16, 16)
                v = accum[i, sl]
                m = v > jnp.full((16,), THRESH, jnp.float32)
                r = jnp.where(m, v + b2v[sl], zf)
                if relu:
                    r = jnp.maximum(r, zf)
                accum[i, sl] = r
            return 0

        lax.fori_loop(0, NPT, post, 0)
        pltpu.sync_copy(accum.at[pl.ds(0, NPT)], h_hbm.at[pl.ds(base, NPT)])

    f = pl.kernel(
        body,
        out_type=jax.ShapeDtypeStruct((N, H), jnp.float32),
        mesh=_mesh(),
        compiler_params=pltpu.CompilerParams(needs_layout_passes=False, use_tc_tiling_on_sc=False),
        scratch_types=[
            pltpu.VMEM((48,), jnp.int32),
            pltpu.VMEM((H,), jnp.float32),
            pltpu.VMEM((QC + 16,), jnp.int32),
            pltpu.VMEM((QC + 16,), jnp.int32),
            pltpu.VMEM((QC // 4, 4 * H), jnp.float32),
            pltpu.VMEM((QC // 4, 4 * H), jnp.float32),
            pltpu.VMEM((NPT + 8, H), jnp.float32),
            pltpu.SemaphoreType.DMA,
            pltpu.SemaphoreType.DMA,
        ],
    )
    return f(Q, bdst, starts48, b2)


def _pool_head(batch, h, pW1, pb1, pW2, pb2, hW1, hb1, hW2, hb2):
    BN = 10000
    GRID = N // BN
    G = 64

    def body(b_ref, h_ref, pw1, pb1r, pw2, pb2r, hw1, hb1r, hw2, hb2r,
             out_ref, sums, cnts):
        i = pl.program_id(0)

        @pl.when(i == 0)
        def _():
            sums[...] = jnp.zeros_like(sums)
            cnts[...] = jnp.zeros_like(cnts)

        b = b_ref[0]
        oh = (b == lax.broadcasted_iota(jnp.int32, (1, G), 1)
              ).astype(jnp.float32)
        hb = h_ref[...]
        sums[...] += lax.dot_general(oh, hb, (((0,), (0,)), ((), ())),
                                     preferred_element_type=jnp.float32)
        cnts[...] += jnp.sum(oh, axis=0, keepdims=True)

        @pl.when(i == GRID - 1)
        def _():
            cnt = jnp.maximum(cnts[...], 1.0)
            mean = sums[...] / cnt.reshape(G, 1)
            z = jnp.maximum(
                jnp.dot(mean, pw1[...], preferred_element_type=jnp.float32)
                + pb1r[...], 0.0)
            z = (jnp.dot(z, pw2[...], preferred_element_type=jnp.float32)
                 + pb2r[...])
            zp = jnp.maximum(
                jnp.dot(z, hw1[...], preferred_element_type=jnp.float32)
                + hb1r[...], 0.0)
            out_ref[...] = (
                jnp.dot(zp, hw2[...], preferred_element_type=jnp.float32)
                + hb2r[...])

    batch3 = batch.reshape(GRID, BN, 1)
    full = lambda shape: pl.BlockSpec(shape, lambda i: tuple(0 for _ in shape))
    return pl.pallas_call(
        body,
        grid=(GRID,),
        in_specs=[
            pl.BlockSpec((1, BN, 1), lambda i: (i, 0, 0)),
            pl.BlockSpec((BN, H), lambda i: (i, 0)),
            full((H, 2 * H)), full((1, 2 * H)),
            full((2 * H, 64)), full((1, 64)),
            full((64, 64)), full((1, 64)),
            full((64, 32)), full((1, 32)),
        ],
        out_specs=pl.BlockSpec((G, 32), lambda i: (0, 0)),
        out_shape=jax.ShapeDtypeStruct((G, 32), jnp.float32),
        scratch_shapes=[
            pltpu.VMEM((G, H), jnp.float32),
            pltpu.VMEM((1, G), jnp.float32),
        ],
    )(batch3, h, pW1, pb1.reshape(1, -1), pW2, pb2.reshape(1, -1),
      hW1, hb1.reshape(1, -1), hW2, hb2.reshape(1, -1))


def kernel(x, edge_index, batch,
           c1_W1, c1_b1, c1_W2, c1_b2,
           c2_W1, c2_b1, c2_W2, c2_b2,
           c3_W1, c3_b1, c3_W2, c3_b2,
           proj_W1, proj_b1, proj_W2, proj_b2,
           head_W1, head_b1, head_W2, head_b2):
    src = edge_index[0]
    dst = edge_index[1]
    ge_counts = _count_edges(dst).reshape(NT, NT, 16).sum(
        axis=(0, 2)).astype(jnp.int32)
    counts = ge_counts - jnp.concatenate(
        [ge_counts[1:], jnp.zeros((1,), jnp.int32)])
    caps = ((counts + (QUANT - 1)) // QUANT) * QUANT
    starts = jnp.concatenate(
        [jnp.zeros((1,), jnp.int32), jnp.cumsum(caps)]).astype(jnp.int32)
    starts48 = jnp.pad(starts, (0, 48 - starts.shape[0]))
    bsrc, bdst = _bucket_edges(src, dst, starts48)

    h = x
    for (W1, b1, W2, b2, relu) in (
            (c1_W1, c1_b1, c1_W2, c1_b2, True),
            (c2_W1, c2_b1, c2_W2, c2_b2, True),
            (c3_W1, c3_b1, c3_W2, c3_b2, False)):
        A, B = _tables(h, W1, b1)
        P = _phase1(A, B, bsrc, bdst, starts48)
        Q = _edge_mlp(P, W2)
        h = _phase2(Q, bdst, starts48, b2, relu)

    return _pool_head(batch, h,
                      proj_W1, proj_b1, proj_W2, proj_b2,
                      head_W1, head_b1, head_W2, head_b2)


# speedup vs baseline: 4.8769x; 1.2563x over previous
import functools

import jax
import jax.numpy as jnp
from jax import lax
from jax.experimental import pallas as pl
from jax.experimental.pallas import tpu as pltpu
from jax.experimental.pallas import tpu_sc as plsc

N = 100000
E = 1600000
H = 32
NT = 32
NPT = N // NT
FC = 4000
QUANT = 1024
RING = 8192
GC = 128
EP = E + NT * QUANT
BM = 1536
NEG = -3.0e38
THRESH = -1.0e38

_mesh = functools.partial(
    plsc.VectorSubcoreMesh, core_axis_name="c", subcore_axis_name="s")


def _wid():
    return lax.axis_index("s") * 2 + lax.axis_index("c")


def _count_edges(dst):
    EPT = E // NT
    CH = 2000
    NCH = EPT // CH

    def body(dst_hbm, cnt_hbm, dbuf0, dbuf1, hist, sem0, sem1):
        wid = _wid()
        ebase = wid * EPT
        bufs = (dbuf0, dbuf1)
        sems = (sem0, sem1)
        for b in range(NT):
            hist[pl.ds(b * 16, 16)] = jnp.zeros((16,), jnp.int32)
        pltpu.async_copy(
            dst_hbm.at[pl.ds(pl.multiple_of(ebase, 8), CH)], dbuf0, sem0)

        def process(buf):
            def vec(vi, _):
                d = buf[pl.ds(vi * 16, 16)]
                for b in range(NT):
                    m = d >= jnp.full((16,), b * NPT, jnp.int32)
                    sl = pl.ds(b * 16, 16)
                    hist[sl] = hist[sl] + jnp.where(m, 1, 0)
                return 0

            lax.fori_loop(0, CH // 16, vec, 0)

        def step(ci, _):
            for par in range(2):
                @pl.when((ci & 1) == par)
                def _():
                    pltpu.make_async_copy(
                        dst_hbm.at[pl.ds(0, CH)], bufs[par], sems[par]).wait()

                    @pl.when(ci + 1 < NCH)
                    def _():
                        off = pl.multiple_of(ebase + (ci + 1) * CH, 8)
                        pltpu.async_copy(dst_hbm.at[pl.ds(off, CH)],
                                         bufs[1 - par], sems[1 - par])

                    process(bufs[par])
            return 0

        lax.fori_loop(0, NCH, step, 0)
        pltpu.sync_copy(hist, cnt_hbm.at[wid])

    f = pl.kernel(
        body,
        out_type=jax.ShapeDtypeStruct((NT, NT * 16), jnp.int32),
        mesh=_mesh(),
        compiler_params=pltpu.CompilerParams(needs_layout_passes=False, use_tc_tiling_on_sc=False),
        scratch_types=[
            pltpu.VMEM((CH,), jnp.int32),
            pltpu.VMEM((CH,), jnp.int32),
            pltpu.VMEM((NT * 16,), jnp.int32),
            pltpu.SemaphoreType.DMA,
            pltpu.SemaphoreType.DMA,
        ],
    )
    return f(dst)


def _bucket_edges(src, dst, starts48):
    def body(src_hbm, dst_hbm, starts_hbm, bsrc_hbm, bdst_hbm,
             sbuf, dbuf, sbuf1, dbuf1, rings, ringd, stv, sem0, sem1):
        wid = _wid()
        base = wid * NPT
        lo = jnp.full((16,), base, jnp.int32)
        hi = jnp.full((16,), base + NPT, jnp.int32)
        sent = jnp.full((16,), base + NPT, jnp.int32)
        zero16 = jnp.zeros((16,), jnp.int32)
        lane = lax.iota(jnp.int32, 16)
        pltpu.sync_copy(starts_hbm, stv)
        st = stv[pl.ds(wid, 16)][0]

        def flush_while(cur, flushed):
            def cond(f):
                return cur - f >= QUANT

            def fbody(f):
                off = pl.multiple_of(f & (RING - 1), QUANT)
                dsto = pl.multiple_of(st + f, QUANT)
                pltpu.sync_copy(ringd.at[pl.ds(off, QUANT)],
                                bdst_hbm.at[pl.ds(dsto, QUANT)])
                pltpu.sync_copy(rings.at[pl.ds(off, QUANT)],
                                bsrc_hbm.at[pl.ds(dsto, QUANT)])
                return f + QUANT

            return lax.while_loop(cond, fbody, flushed)

        def chunk(ci, carry):
            curv, flushed = carry
            co = pl.multiple_of(ci * FC, FC)
            pltpu.sync_copy(src_hbm.at[pl.ds(co, FC)], sbuf)
            pltpu.sync_copy(dst_hbm.at[pl.ds(co, FC)], dbuf)

            def vec(vi, cv):
                d = dbuf[pl.ds(vi * 16, 16)]
                s = sbuf[pl.ds(vi * 16, 16)]
                m = (d >= lo) & (d < hi)
                csum = plsc.cumsum(jnp.where(m, 1, 0))
                pos = (cv + csum - 1) & (RING - 1)
                plsc.store_scatter(ringd, [pos], d, mask=m)
                plsc.store_scatter(rings, [pos], s, mask=m)
                return cv + csum[15]

            curv = lax.fori_loop(0, FC // 16, vec, curv)
            flushed = flush_while(curv[0], flushed)
            return curv, flushed

        curv, flushed = lax.fori_loop(
            0, E // FC, chunk, (jnp.zeros((16,), jnp.int32), jnp.int32(0)))

        tgt = ((curv + (QUANT - 1)) >> 10) << 10
        for j in range(QUANT // 16):
            pos = curv + j * 16 + lane
            m = pos < tgt
            plsc.store_scatter(ringd, [pos & (RING - 1)], sent, mask=m)
            plsc.store_scatter(rings, [pos & (RING - 1)], zero16, mask=m)
        flushed = flush_while(tgt[0], flushed)

    f = pl.kernel(
        body,
        out_type=(jax.ShapeDtypeStruct((EP,), jnp.int32),
                  jax.ShapeDtypeStruct((EP,), jnp.int32)),
        mesh=_mesh(),
        compiler_params=pltpu.CompilerParams(needs_layout_passes=False, use_tc_tiling_on_sc=False),
        scratch_types=[
            pltpu.VMEM((FC,), jnp.int32),
            pltpu.VMEM((FC,), jnp.int32),
            pltpu.VMEM((FC,), jnp.int32),
            pltpu.VMEM((FC,), jnp.int32),
            pltpu.VMEM((RING,), jnp.int32),
            pltpu.VMEM((RING,), jnp.int32),
            pltpu.VMEM((48,), jnp.int32),
            pltpu.SemaphoreType.DMA,
            pltpu.SemaphoreType.DMA,
        ],
    )
    return f(src, dst, starts48)


def _tables(h, W1, b1):
    F = h.shape[1]
    BN = 10000

    def body(x_ref, w_ref, b_ref, a_ref, bb_ref):
        xb = x_ref[...]
        w = w_ref[...]
        wa = w[:F, :]
        wb = w[F:, :]
        bb_ref[...] = jnp.dot(xb, wb, preferred_element_type=jnp.float32)
        a_ref[...] = (jnp.dot(xb, wa - wb, preferred_element_type=jnp.float32)
                      + b_ref[...])

    return pl.pallas_call(
        body,
        grid=(N // BN,),
        in_specs=[
            pl.BlockSpec((BN, F), lambda i: (i, 0)),
            pl.BlockSpec((2 * F, H), lambda i: (0, 0)),
            pl.BlockSpec((1, H), lambda i: (0, 0)),
        ],
        out_specs=[
            pl.BlockSpec((BN, H), lambda i: (i, 0)),
            pl.BlockSpec((BN, H), lambda i: (i, 0)),
        ],
        out_shape=[jax.ShapeDtypeStruct((N, H), jnp.float32)] * 2,
    )(h, W1, b1.reshape(1, H))


SUP = 512
NG = SUP // GC


def _phase1(A, B, bsrc, bdst, starts48):
    def body(a_hbm, b_hbm, bsrc_hbm, bdst_hbm, starts_hbm, p_hbm,
             stv, d0, s0, d1, s1, ar0, br0, ar1, br1, pb0, pb1,
             sg0, sg1, si0, si1, pw0, pw1):
        wid = _wid()
        pltpu.sync_copy(starts_hbm, stv)
        sv = stv[pl.ds(wid, 16)]
        st = sv[0]
        nch = (sv[1] - st) >> 9
        nmax = jnp.full((16,), N - 1, jnp.int32)
        zf = jnp.zeros((16,), jnp.float32)
        ibd = (d0, d1)
        ibs = (s0, s1)
        ars = (ar0, ar1)
        brs = (br0, br1)
        pbs = (pb0, pb1)
        sg = (sg0, sg1)
        si = (si0, si1)
        pw = (pw0, pw1)

        def off_of(ci):
            return pl.multiple_of(st + ci * SUP, GC)

        def issue_idx(ci, b):
            off = off_of(ci)
            pltpu.async_copy(bdst_hbm.at[pl.ds(off, SUP)], ibd[b], si[b])
            pltpu.async_copy(bsrc_hbm.at[pl.ds(off, SUP)], ibs[b], si[b])

        def wait_idx(b):
            pltpu.make_async_copy(
                bdst_hbm.at[pl.ds(0, SUP)], ibd[b], si[b]).wait()
            pltpu.make_async_copy(
                bsrc_hbm.at[pl.ds(0, SUP)], ibs[b], si[b]).wait()

        def issue_gathers(b):
            for v in range(SUP // 16):
                sl = pl.ds(v * 16, 16)
                ibd[b][sl] = jnp.minimum(ibd[b][sl], nmax)
            for g in range(NG):
                gs = pl.ds(g * GC, GC)
                pltpu.async_copy(a_hbm.at[ibd[b].at[gs]], ars[b].at[gs], sg[b])
                pltpu.async_copy(b_hbm.at[ibs[b].at[gs]], brs[b].at[gs], sg[b])

        def wait_gathers(b):
            for g in range(2 * NG):
                pltpu.make_async_copy(
                    p_hbm.at[pl.ds(0, GC)], ars[b].at[pl.ds(0, GC)],
                    sg[b]).wait()

        def compute(ci, b):
            def row(r0, _):
                for u in range(4):
                    r = r0 * 4 + u
                    for hh in range(2):
                        sl = pl.ds(hh * 16, 16)
                        po = pl.ds(u * 32 + hh * 16, 16)
                        pbs[b][r0, po] = jnp.maximum(
                            ars[b][r, sl] + brs[b][r, sl], zf)
                return 0

            lax.fori_loop(0, SUP // 4, row, 0)
            offq = pl.multiple_of((st + ci * SUP) >> 2, SUP // 4)
            pltpu.async_copy(pbs[b], p_hbm.at[pl.ds(offq, SUP // 4)], pw[b])

        def wait_pw(b):
            pltpu.make_async_copy(
                p_hbm.at[pl.ds(0, SUP // 4)], pbs[b], pw[b]).wait()

        @pl.when(nch > 0)
        def _():
            issue_idx(0, 0)
            wait_idx(0)
            issue_gathers(0)
            issue_idx(1, 1)

            def step(ci, _):
                for par in range(2):
                    @pl.when((ci & 1) == par)
                    def _():
                        wait_gathers(par)

                        @pl.when(ci + 1 < nch)
                        def _():
                            wait_idx(1 - par)
                            issue_gathers(1 - par)

                        @pl.when(ci + 2 < nch)
                        def _():
                            issue_idx(ci + 2, par)

                        @pl.when(ci >= 2)
                        def _():
                            wait_pw(par)

                        compute(ci, par)
                return 0

            lax.fori_loop(0, nch, step, 0)
            wait_pw(0)
            wait_pw(1)

    f = pl.kernel(
        body,
        out_type=jax.ShapeDtypeStruct((EP // 4, 4 * H), jnp.float32),
        mesh=_mesh(),
        compiler_params=pltpu.CompilerParams(needs_layout_passes=False, use_tc_tiling_on_sc=False),
        scratch_types=[
            pltpu.VMEM((48,), jnp.int32),
            pltpu.VMEM((SUP,), jnp.int32),
            pltpu.VMEM((SUP,), jnp.int32),
            pltpu.VMEM((SUP,), jnp.int32),
            pltpu.VMEM((SUP,), jnp.int32),
            pltpu.VMEM((SUP, H), jnp.float32),
            pltpu.VMEM((SUP, H), jnp.float32),
            pltpu.VMEM((SUP, H), jnp.float32),
            pltpu.VMEM((SUP, H), jnp.float32),
            pltpu.VMEM((SUP // 4, 4 * H), jnp.float32),
            pltpu.VMEM((SUP // 4, 4 * H), jnp.float32),
            pltpu.SemaphoreType.DMA,
            pltpu.SemaphoreType.DMA,
            pltpu.SemaphoreType.DMA,
            pltpu.SemaphoreType.DMA,
            pltpu.SemaphoreType.DMA,
            pltpu.SemaphoreType.DMA,
        ],
    )
    return f(A, B, bsrc, bdst, starts48)


def _edge_mlp(P, W2):
    EPQ = EP // 4
    BMQ = 384

    def body(p_ref, w_ref, q_ref):
        w = w_ref[...]
        z = jnp.zeros((H, H), jnp.float32)
        wd = jnp.concatenate(
            [jnp.concatenate([w if j == i else z for j in range(4)], axis=1)
             for i in range(4)], axis=0)
        q_ref[...] = jnp.dot(p_ref[...], wd,
                             preferred_element_type=jnp.float32)

    return pl.pallas_call(
        body,
        grid=(EPQ // BMQ,),
        in_specs=[
            pl.BlockSpec((BMQ, 4 * H), lambda i: (i, 0)),
            pl.BlockSpec((H, H), lambda i: (0, 0)),
        ],
        out_specs=pl.BlockSpec((BMQ, 4 * H), lambda i: (i, 0)),
        out_shape=jax.ShapeDtypeStruct((EPQ, 4 * H), jnp.float32),
    )(P, W2)


def _phase2(Q, bdst, starts48, b2, relu):
    QC = 256

    def body(q_hbm, bdst_hbm, starts_hbm, b2_hbm, h_hbm,
             stv, b2v, db0, db1, qb0, qb1, accum, sq0, sq1):
        wid = _wid()
        base = wid * NPT
        pltpu.sync_copy(starts_hbm, stv)
        pltpu.sync_copy(b2_hbm, b2v)
        sv = stv[pl.ds(wid, 16)]
        st = sv[0]
        nch = (sv[1] - st) >> 8
        neg = jnp.full((16,), NEG, jnp.float32)
        basev = jnp.full((16,), base, jnp.int32)
        zf = jnp.zeros((16,), jnp.float32)
        dbs = (db0, db1)
        qbs = (qb0, qb1)
        sq = (sq0, sq1)

        def init(i, _):
            for u in range(4):
                accum[i * 4 + u, pl.ds(0, 16)] = neg
                accum[i * 4 + u, pl.ds(16, 16)] = neg
            return 0

        lax.fori_loop(0, (NPT + 4) // 4, init, 0)

        def issue(ci, b):
            off = pl.multiple_of(st + ci * QC, QC)
            offq = pl.multiple_of((st + ci * QC) >> 2, QC // 4)
            pltpu.async_copy(q_hbm.at[pl.ds(offq, QC // 4)], qbs[b], sq[b])
            pltpu.async_copy(bdst_hbm.at[pl.ds(off, QC)],
                             dbs[b].at[pl.ds(0, QC)], sq[b])

        def wait_io(b):
            pltpu.make_async_copy(q_hbm.at[pl.ds(0, QC // 4)], qbs[b],
                                  sq[b]).wait()
            pltpu.make_async_copy(bdst_hbm.at[pl.ds(0, QC)],
                                  dbs[b].at[pl.ds(0, QC)], sq[b]).wait()

        def process(b):
            dbuf = dbs[b]
            qbuf = qbs[b]
            for v in range(QC // 16):
                sl = pl.ds(v * 16, 16)
                dbuf[sl] = dbuf[sl] - basev

            def edge16(e0, _):
                dv = dbuf[pl.ds(e0 * 16, 16)]
                for u in range(16):
                    dl = dv[u]
                    qr = e0 * 4 + u // 4
                    for hh in range(2):
                        sl = pl.ds(hh * 16, 16)
                        qo = pl.ds((u % 4) * 32 + hh * 16, 16)
                        accum[dl, sl] = jnp.maximum(accum[dl, sl],
                                                    qbuf[qr, qo])
                return 0

            lax.fori_loop(0, QC // 16, edge16, 0)

        @pl.when(nch > 0)
        def _():
            issue(0, 0)

            def step(ci, _):
                for par in range(2):
                    @pl.when((ci & 1) == par)
                    def _():
                        wait_io(par)

                        @pl.when(ci + 1 < nch)
                        def _():
                            issue(ci + 1, 1 - par)

                        process(par)
                return 0

            lax.fori_loop(0, nch, step, 0)

        def post(i, _):
            for hh in range(2):
                sl = pl.ds(hh * 16, 16)
                v = accum[i, sl]
                m = v > jnp.full((16,), THRESH, jnp.float32)
                r = jnp.where(m, v + b2v[sl], zf)
                if relu:
                    r = jnp.maximum(r, zf)
                accum[i, sl] = r
            return 0

        lax.fori_loop(0, NPT, post, 0)
        pltpu.sync_copy(accum.at[pl.ds(0, NPT)], h_hbm.at[pl.ds(base, NPT)])

    f = pl.kernel(
        body,
        out_type=jax.ShapeDtypeStruct((N, H), jnp.float32),
        mesh=_mesh(),
        compiler_params=pltpu.CompilerParams(needs_layout_passes=False, use_tc_tiling_on_sc=False),
        scratch_types=[
            pltpu.VMEM((48,), jnp.int32),
            pltpu.VMEM((H,), jnp.float32),
            pltpu.VMEM((QC + 16,), jnp.int32),
            pltpu.VMEM((QC + 16,), jnp.int32),
            pltpu.VMEM((QC // 4, 4 * H), jnp.float32),
            pltpu.VMEM((QC // 4, 4 * H), jnp.float32),
            pltpu.VMEM((NPT + 8, H), jnp.float32),
            pltpu.SemaphoreType.DMA,
            pltpu.SemaphoreType.DMA,
        ],
    )
    return f(Q, bdst, starts48, b2)


def _pool_head(batch, h, pW1, pb1, pW2, pb2, hW1, hb1, hW2, hb2):
    BN = 10000
    GRID = N // BN
    G = 64

    def body(b_ref, h_ref, pw1, pb1r, pw2, pb2r, hw1, hb1r, hw2, hb2r,
             out_ref, sums, cnts):
        i = pl.program_id(0)

        @pl.when(i == 0)
        def _():
            sums[...] = jnp.zeros_like(sums)
            cnts[...] = jnp.zeros_like(cnts)

        b = b_ref[0]
        oh = (b == lax.broadcasted_iota(jnp.int32, (1, G), 1)
              ).astype(jnp.float32)
        hb = h_ref[...]
        sums[...] += lax.dot_general(oh, hb, (((0,), (0,)), ((), ())),
                                     preferred_element_type=jnp.float32)
        cnts[...] += jnp.sum(oh, axis=0, keepdims=True)

        @pl.when(i == GRID - 1)
        def _():
            cnt = jnp.maximum(cnts[...], 1.0)
            mean = sums[...] / cnt.reshape(G, 1)
            z = jnp.maximum(
                jnp.dot(mean, pw1[...], preferred_element_type=jnp.float32)
                + pb1r[...], 0.0)
            z = (jnp.dot(z, pw2[...], preferred_element_type=jnp.float32)
                 + pb2r[...])
            zp = jnp.maximum(
                jnp.dot(z, hw1[...], preferred_element_type=jnp.float32)
                + hb1r[...], 0.0)
            out_ref[...] = (
                jnp.dot(zp, hw2[...], preferred_element_type=jnp.float32)
                + hb2r[...])

    batch3 = batch.reshape(GRID, BN, 1)
    full = lambda shape: pl.BlockSpec(shape, lambda i: tuple(0 for _ in shape))
    return pl.pallas_call(
        body,
        grid=(GRID,),
        in_specs=[
            pl.BlockSpec((1, BN, 1), lambda i: (i, 0, 0)),
            pl.BlockSpec((BN, H), lambda i: (i, 0)),
            full((H, 2 * H)), full((1, 2 * H)),
            full((2 * H, 64)), full((1, 64)),
            full((64, 64)), full((1, 64)),
            full((64, 32)), full((1, 32)),
        ],
        out_specs=pl.BlockSpec((G, 32), lambda i: (0, 0)),
        out_shape=jax.ShapeDtypeStruct((G, 32), jnp.float32),
        scratch_shapes=[
            pltpu.VMEM((G, H), jnp.float32),
            pltpu.VMEM((1, G), jnp.float32),
        ],
    )(batch3, h, pW1, pb1.reshape(1, -1), pW2, pb2.reshape(1, -1),
      hW1, hb1.reshape(1, -1), hW2, hb2.reshape(1, -1))


def kernel(x, edge_index, batch,
           c1_W1, c1_b1, c1_W2, c1_b2,
           c2_W1, c2_b1, c2_W2, c2_b2,
           c3_W1, c3_b1, c3_W2, c3_b2,
           proj_W1, proj_b1, proj_W2, proj_b2,
           head_W1, head_b1, head_W2, head_b2):
    src = edge_index[0]
    dst = edge_index[1]
    ge_counts = _count_edges(dst).reshape(NT, NT, 16).sum(
        axis=(0, 2)).astype(jnp.int32)
    counts = ge_counts - jnp.concatenate(
        [ge_counts[1:], jnp.zeros((1,), jnp.int32)])
    caps = ((counts + (QUANT - 1)) // QUANT) * QUANT
    starts = jnp.concatenate(
        [jnp.zeros((1,), jnp.int32), jnp.cumsum(caps)]).astype(jnp.int32)
    starts48 = jnp.pad(starts, (0, 48 - starts.shape[0]))
    bsrc, bdst = _bucket_edges(src, dst, starts48)

    h = x
    for (W1, b1, W2, b2, relu) in (
            (c1_W1, c1_b1, c1_W2, c1_b2, True),
            (c2_W1, c2_b1, c2_W2, c2_b2, True),
            (c3_W1, c3_b1, c3_W2, c3_b2, False)):
        A, B = _tables(h, W1, b1)
        P = _phase1(A, B, bsrc, bdst, starts48)
        Q = _edge_mlp(P, W2)
        h = _phase2(Q, bdst, starts48, b2, relu)

    return _pool_head(batch, h,
                      proj_W1, proj_b1, proj_W2, proj_b2,
                      head_W1, head_b1, head_W2, head_b2)
